# Initial kernel scaffold; baseline (speedup 1.0000x reference)
import jax
import jax.numpy as jnp
from jax.experimental import pallas as pl


def kernel(x, edge_index, u, batch, W1, b1, W2, b2, W3, b3):
    raise NotImplementedError("write your pallas kernel here")


# speedup vs baseline: 10.2712x; 10.2712x over previous
import functools

import jax
import jax.numpy as jnp
from jax import lax
from jax.experimental import pallas as pl
from jax.experimental.pallas import tpu as pltpu

N = 10000
D_NODE = 256
D_GLOBAL = 128
B = 64
HIDDEN = 512
D_OUT = 256

R = 1000
G = N // R


def _fused_body(x_ref, b_ref, u_ref, w1_ref, b1_ref, w2_ref, b2_ref,
                w3_ref, b3_ref, o_ref, acc_ref):
    i = pl.program_id(0)

    @pl.when(i == 0)
    def _():
        acc_ref[:] = jnp.zeros_like(acc_ref)

    seg = b_ref[0]
    onehot_t = jnp.equal(
        lax.broadcasted_iota(jnp.int32, (B, R), 0), seg
    ).astype(jnp.float32)
    acc_ref[:] += lax.dot_general(
        onehot_t, x_ref[:], (((1,), (0,)), ((), ())),
        preferred_element_type=jnp.float32)

    @pl.when(i == G - 1)
    def _():
        h0 = jnp.concatenate([acc_ref[:], u_ref[:]], axis=1)
        dn = (((1,), (1,)), ((), ()))
        h1 = jnp.maximum(
            lax.dot_general(h0, w1_ref[:], dn,
                            preferred_element_type=jnp.float32)
            + b1_ref[:], 0.0)
        h2 = jnp.maximum(
            lax.dot_general(h1, w2_ref[:], dn,
                            preferred_element_type=jnp.float32)
            + b2_ref[:], 0.0)
        o_ref[:] = (
            lax.dot_general(h2, w3_ref[:], dn,
                            preferred_element_type=jnp.float32)
            + b3_ref[:])


_fused = pl.pallas_call(
    _fused_body,
    grid=(G,),
    in_specs=[
        pl.BlockSpec((R, D_NODE), lambda i: (i, 0)),
        pl.BlockSpec((1, 1, R), lambda i: (i, 0, 0)),
        pl.BlockSpec((B, D_GLOBAL), lambda i: (0, 0)),
        pl.BlockSpec((HIDDEN, D_NODE + D_GLOBAL), lambda i: (0, 0)),
        pl.BlockSpec((1, HIDDEN), lambda i: (0, 0)),
        pl.BlockSpec((HIDDEN, HIDDEN), lambda i: (0, 0)),
        pl.BlockSpec((1, HIDDEN), lambda i: (0, 0)),
        pl.BlockSpec((D_OUT, HIDDEN), lambda i: (0, 0)),
        pl.BlockSpec((1, D_OUT), lambda i: (0, 0)),
    ],
    out_specs=pl.BlockSpec((B, D_OUT), lambda i: (0, 0)),
    out_shape=jax.ShapeDtypeStruct((B, D_OUT), jnp.float32),
    scratch_shapes=[pltpu.VMEM((B, D_NODE), jnp.float32)],
)


@jax.jit
def kernel(x, edge_index, u, batch, W1, b1, W2, b2, W3, b3):
    del edge_index
    return _fused(x, batch.reshape(G, 1, R), u,
                  W1, b1.reshape(1, HIDDEN),
                  W2, b2.reshape(1, HIDDEN),
                  W3, b3.reshape(1, D_OUT))


# speedup vs baseline: 12.6952x; 1.2360x over previous
import functools

import jax
import jax.numpy as jnp
from jax import lax
from jax.experimental import pallas as pl
from jax.experimental.pallas import tpu as pltpu

N = 10000
D_NODE = 256
D_GLOBAL = 128
B = 64
HIDDEN = 512
D_OUT = 256

R = 2000
G = N // R


def _fused_body(x_ref, b_ref, u_ref, w1_ref, b1_ref, w2_ref, b2_ref,
                w3_ref, b3_ref, o_ref, acc_ref):
    i = pl.program_id(0)

    @pl.when(i == 0)
    def _():
        acc_ref[:] = jnp.zeros_like(acc_ref)

    seg = b_ref[0]
    onehot_t = jnp.equal(
        lax.broadcasted_iota(jnp.int32, (B, R), 0), seg
    ).astype(jnp.float32)
    acc_ref[:] += lax.dot_general(
        onehot_t, x_ref[:], (((1,), (0,)), ((), ())),
        preferred_element_type=jnp.float32)

    @pl.when(i == G - 1)
    def _():
        h0 = jnp.concatenate([acc_ref[:], u_ref[:]], axis=1)
        dn = (((1,), (1,)), ((), ()))
        h1 = jnp.maximum(
            lax.dot_general(h0, w1_ref[:], dn,
                            preferred_element_type=jnp.float32)
            + b1_ref[:], 0.0)
        h2 = jnp.maximum(
            lax.dot_general(h1, w2_ref[:], dn,
                            preferred_element_type=jnp.float32)
            + b2_ref[:], 0.0)
        o_ref[:] = (
            lax.dot_general(h2, w3_ref[:], dn,
                            preferred_element_type=jnp.float32)
            + b3_ref[:])


_fused = pl.pallas_call(
    _fused_body,
    grid=(G,),
    in_specs=[
        pl.BlockSpec((R, D_NODE), lambda i: (i, 0)),
        pl.BlockSpec((1, 1, R), lambda i: (i, 0, 0)),
        pl.BlockSpec((B, D_GLOBAL), lambda i: (0, 0)),
        pl.BlockSpec((HIDDEN, D_NODE + D_GLOBAL), lambda i: (0, 0)),
        pl.BlockSpec((1, HIDDEN), lambda i: (0, 0)),
        pl.BlockSpec((HIDDEN, HIDDEN), lambda i: (0, 0)),
        pl.BlockSpec((1, HIDDEN), lambda i: (0, 0)),
        pl.BlockSpec((D_OUT, HIDDEN), lambda i: (0, 0)),
        pl.BlockSpec((1, D_OUT), lambda i: (0, 0)),
    ],
    out_specs=pl.BlockSpec((B, D_OUT), lambda i: (0, 0)),
    out_shape=jax.ShapeDtypeStruct((B, D_OUT), jnp.float32),
    scratch_shapes=[pltpu.VMEM((B, D_NODE), jnp.float32)],
)


@jax.jit
def kernel(x, edge_index, u, batch, W1, b1, W2, b2, W3, b3):
    del edge_index
    return _fused(x, batch.reshape(G, 1, R), u,
                  W1, b1.reshape(1, HIDDEN),
                  W2, b2.reshape(1, HIDDEN),
                  W3, b3.reshape(1, D_OUT))


# speedup vs baseline: 14.8631x; 1.1708x over previous
import functools

import jax
import jax.numpy as jnp
from jax import lax
from jax.experimental import pallas as pl
from jax.experimental.pallas import tpu as pltpu

N = 10000
D_NODE = 256
D_GLOBAL = 128
B = 64
HIDDEN = 512
D_OUT = 256

R = 5000
G = N // R


def _fused_body(x_ref, b_ref, u_ref, w1_ref, b1_ref, w2_ref, b2_ref,
                w3_ref, b3_ref, o_ref, acc_ref):
    i = pl.program_id(0)

    @pl.when(i == 0)
    def _():
        acc_ref[:] = jnp.zeros_like(acc_ref)

    seg = b_ref[0]
    onehot_t = jnp.equal(
        lax.broadcasted_iota(jnp.int32, (B, R), 0), seg
    ).astype(jnp.float32)
    acc_ref[:] += lax.dot_general(
        onehot_t, x_ref[:], (((1,), (0,)), ((), ())),
        preferred_element_type=jnp.float32)

    @pl.when(i == G - 1)
    def _():
        h0 = jnp.concatenate([acc_ref[:], u_ref[:]], axis=1)
        dn = (((1,), (1,)), ((), ()))
        h1 = jnp.maximum(
            lax.dot_general(h0, w1_ref[:], dn,
                            preferred_element_type=jnp.float32)
            + b1_ref[:], 0.0)
        h2 = jnp.maximum(
            lax.dot_general(h1, w2_ref[:], dn,
                            preferred_element_type=jnp.float32)
            + b2_ref[:], 0.0)
        o_ref[:] = (
            lax.dot_general(h2, w3_ref[:], dn,
                            preferred_element_type=jnp.float32)
            + b3_ref[:])


_fused = pl.pallas_call(
    _fused_body,
    grid=(G,),
    in_specs=[
        pl.BlockSpec((R, D_NODE), lambda i: (i, 0)),
        pl.BlockSpec((1, 1, R), lambda i: (i, 0, 0)),
        pl.BlockSpec((B, D_GLOBAL), lambda i: (0, 0)),
        pl.BlockSpec((HIDDEN, D_NODE + D_GLOBAL), lambda i: (0, 0)),
        pl.BlockSpec((1, HIDDEN), lambda i: (0, 0)),
        pl.BlockSpec((HIDDEN, HIDDEN), lambda i: (0, 0)),
        pl.BlockSpec((1, HIDDEN), lambda i: (0, 0)),
        pl.BlockSpec((D_OUT, HIDDEN), lambda i: (0, 0)),
        pl.BlockSpec((1, D_OUT), lambda i: (0, 0)),
    ],
    out_specs=pl.BlockSpec((B, D_OUT), lambda i: (0, 0)),
    out_shape=jax.ShapeDtypeStruct((B, D_OUT), jnp.float32),
    scratch_shapes=[pltpu.VMEM((B, D_NODE), jnp.float32)],
)


@jax.jit
def kernel(x, edge_index, u, batch, W1, b1, W2, b2, W3, b3):
    del edge_index
    return _fused(x, batch.reshape(G, 1, R), u,
                  W1, b1.reshape(1, HIDDEN),
                  W2, b2.reshape(1, HIDDEN),
                  W3, b3.reshape(1, D_OUT))
